# baseline (speedup 1.0000x reference)
import functools

import jax
import jax.numpy as jnp
from jax import lax
from jax.experimental import pallas as pl
from jax.experimental.pallas import tpu as pltpu
from jax.experimental.pallas import tpu_sc as plsc

_B = 64
_P = 64
_H = 512
_W = 512

_L = 16
_NC = 1
_NS = 8
_NW = _NC * _NS
_BPW = _B // _NW

_RUNS_PER_B = 2 * _H * _W // 128

_mesh = plsc.VectorSubcoreMesh(core_axis_name="c", subcore_axis_name="s", num_cores=1, num_subcores=8)


@functools.partial(
    pl.kernel,
    mesh=_mesh,
    out_type=jax.ShapeDtypeStruct((_B, _P), jnp.float32),
    scratch_types=[
        pltpu.VMEM((2 * _P,), jnp.int32),
        pltpu.VMEM((_BPW, _P), jnp.int32),
        pltpu.VMEM((_BPW, _P, 128), jnp.float32),
        pltpu.VMEM((_BPW, _P), jnp.float32),
        pltpu.SemaphoreType.DMA,
        pltpu.SemaphoreType.DMA,
        pltpu.SemaphoreType.DMA,
        pltpu.SemaphoreType.DMA,
        pltpu.SemaphoreType.DMA,
        pltpu.SemaphoreType.DMA,
        pltpu.SemaphoreType.DMA,
        pltpu.SemaphoreType.DMA,
        pltpu.SemaphoreType.DMA,
    ],
    compiler_params=pltpu.CompilerParams(needs_layout_passes=False),
)
def _probe_gather(m_hbm, xy_hbm, out_hbm, xy_v, ridx_v, runs_v, val_v,
                  gsem0, gsem1, gsem2, gsem3, gsem4, gsem5, gsem6, gsem7, osem):
    wid = lax.axis_index("s")
    b0 = wid * _BPW
    pltpu.sync_copy(xy_hbm, xy_v)

    gsems = (gsem0, gsem1, gsem2, gsem3, gsem4, gsem5, gsem6, gsem7)
    gathers = []
    for j in range(_BPW):
        for c in range(_P // _L):
            xv = xy_v[pl.ds(c * _L, _L)]
            yv = xy_v[pl.ds(_P + c * _L, _L)]
            run = (xv >> 3) * 32 + (yv >> 7) * 8 + (xv & 7)
            ridx_v[j, pl.ds(c * _L, _L)] = run + (b0 + j) * _RUNS_PER_B
        gathers.append(
            pltpu.async_copy(m_hbm.at[ridx_v.at[j]], runs_v.at[j], gsems[j])
        )

    for j in range(_BPW):
        gathers[j].wait()
        for c in range(_P // _L):
            rids = lax.iota(jnp.int32, _L) + c * _L
            lids = xy_v[pl.ds(_P + c * _L, _L)] & 127
            val_v[j, pl.ds(c * _L, _L)] = plsc.load_gather(
                runs_v.at[j], [rids, lids]
            )
        if j < _BPW - 1:
            pltpu.async_copy(val_v.at[j], out_hbm.at[b0 + j], osem)
    pltpu.sync_copy(val_v.at[_BPW - 1], out_hbm.at[b0 + _BPW - 1])
    for j in range(_BPW - 1):
        pltpu.make_async_copy(val_v.at[j], out_hbm.at[b0 + j], osem).wait()


def kernel(m, x, y):
    mruns = (
        m.reshape(8192, 8, 4, 128)
        .transpose(0, 2, 1, 3)
        .reshape(_B * _RUNS_PER_B, 128)
    )
    xy = jnp.concatenate([x.astype(jnp.int32), y.astype(jnp.int32)])
    return _probe_gather(mruns, xy)


# speedup vs baseline: 1.0630x; 1.0630x over previous
import functools

import jax
import jax.numpy as jnp
from jax import lax
from jax.experimental import pallas as pl
from jax.experimental.pallas import tpu as pltpu
from jax.experimental.pallas import tpu_sc as plsc

_B = 64
_P = 64
_H = 512
_W = 512

_L = 16
_NC = 1
_NS = 16
_NW = _NC * _NS
_BPW = _B // _NW

_RUNS_PER_B = 2 * _H * _W // 128

_mesh = plsc.VectorSubcoreMesh(core_axis_name="c", subcore_axis_name="s", num_cores=1)


@functools.partial(
    pl.kernel,
    mesh=_mesh,
    out_type=jax.ShapeDtypeStruct((_B, _P), jnp.float32),
    scratch_types=[
        pltpu.VMEM((2 * _P,), jnp.int32),
        pltpu.VMEM((_BPW, _P), jnp.int32),
        pltpu.VMEM((_BPW, _P, 128), jnp.float32),
        pltpu.VMEM((_BPW, _P), jnp.float32),
        pltpu.SemaphoreType.DMA,
        pltpu.SemaphoreType.DMA,
        pltpu.SemaphoreType.DMA,
        pltpu.SemaphoreType.DMA,
        pltpu.SemaphoreType.DMA,
    ],
    compiler_params=pltpu.CompilerParams(needs_layout_passes=False),
)
def _probe_gather(m_hbm, xy_hbm, out_hbm, xy_v, ridx_v, runs_v, val_v,
                  gsem0, gsem1, gsem2, gsem3, osem):
    wid = lax.axis_index("s")
    b0 = wid * _BPW
    pltpu.sync_copy(xy_hbm, xy_v)

    gsems = (gsem0, gsem1, gsem2, gsem3)
    gathers = []
    for j in range(_BPW):
        for c in range(_P // _L):
            xv = xy_v[pl.ds(c * _L, _L)]
            yv = xy_v[pl.ds(_P + c * _L, _L)]
            run = (xv >> 3) * 32 + (yv >> 7) * 8 + (xv & 7)
            ridx_v[j, pl.ds(c * _L, _L)] = run + (b0 + j) * _RUNS_PER_B
        gathers.append(
            pltpu.async_copy(m_hbm.at[ridx_v.at[j]], runs_v.at[j], gsems[j])
        )

    for j in range(_BPW):
        gathers[j].wait()
        for c in range(_P // _L):
            rids = lax.iota(jnp.int32, _L) + c * _L
            lids = xy_v[pl.ds(_P + c * _L, _L)] & 127
            val_v[j, pl.ds(c * _L, _L)] = plsc.load_gather(
                runs_v.at[j], [rids, lids]
            )
        if j < _BPW - 1:
            pltpu.async_copy(val_v.at[j], out_hbm.at[b0 + j], osem)
    pltpu.sync_copy(val_v.at[_BPW - 1], out_hbm.at[b0 + _BPW - 1])
    for j in range(_BPW - 1):
        pltpu.make_async_copy(val_v.at[j], out_hbm.at[b0 + j], osem).wait()


def kernel(m, x, y):
    mruns = (
        m.reshape(8192, 8, 4, 128)
        .transpose(0, 2, 1, 3)
        .reshape(_B * _RUNS_PER_B, 128)
    )
    xy = jnp.concatenate([x.astype(jnp.int32), y.astype(jnp.int32)])
    return _probe_gather(mruns, xy)


# speedup vs baseline: 1.0681x; 1.0049x over previous
import functools

import jax
import jax.numpy as jnp
from jax import lax
from jax.experimental import pallas as pl
from jax.experimental.pallas import tpu as pltpu
from jax.experimental.pallas import tpu_sc as plsc

_B = 64
_P = 64
_H = 512
_W = 512

_L = 16
_NC = 1
_NS = 16
_NW = _NC * _NS
_BPW = _B // _NW

_RUNS_PER_B = 2 * _H * _W // 128

_mesh = plsc.VectorSubcoreMesh(core_axis_name="c", subcore_axis_name="s", num_cores=1)


@functools.partial(
    pl.kernel,
    mesh=_mesh,
    out_type=jax.ShapeDtypeStruct((_B, _P), jnp.float32),
    scratch_types=[
        pltpu.VMEM((_P,), jnp.int32),
        pltpu.VMEM((_P,), jnp.int32),
        pltpu.VMEM((_BPW, _P), jnp.int32),
        pltpu.VMEM((_BPW, _P, 128), jnp.float32),
        pltpu.VMEM((_BPW, _P), jnp.float32),
        pltpu.SemaphoreType.DMA,
        pltpu.SemaphoreType.DMA,
        pltpu.SemaphoreType.DMA,
        pltpu.SemaphoreType.DMA,
        pltpu.SemaphoreType.DMA,
    ],
    compiler_params=pltpu.CompilerParams(needs_layout_passes=False),
)
def _probe_gather(m_hbm, x_hbm, y_hbm, out_hbm, x_v, y_v, ridx_v, runs_v, val_v,
                  gsem0, gsem1, gsem2, gsem3, osem):
    wid = lax.axis_index("s")
    b0 = wid * _BPW
    cpx = pltpu.async_copy(x_hbm, x_v, gsem0)
    cpy = pltpu.async_copy(y_hbm, y_v, gsem1)
    cpx.wait()
    cpy.wait()

    gsems = (gsem0, gsem1, gsem2, gsem3)
    gathers = []
    for j in range(_BPW):
        for c in range(_P // _L):
            xv = x_v[pl.ds(c * _L, _L)]
            yv = y_v[pl.ds(c * _L, _L)]
            run = (xv >> 3) * 32 + (yv >> 7) * 8 + (xv & 7)
            ridx_v[j, pl.ds(c * _L, _L)] = run + (b0 + j) * _RUNS_PER_B
        gathers.append(
            pltpu.async_copy(m_hbm.at[ridx_v.at[j]], runs_v.at[j], gsems[j])
        )

    for j in range(_BPW):
        gathers[j].wait()
        for c in range(_P // _L):
            rids = lax.iota(jnp.int32, _L) + c * _L
            lids = y_v[pl.ds(c * _L, _L)] & 127
            val_v[j, pl.ds(c * _L, _L)] = plsc.load_gather(
                runs_v.at[j], [rids, lids]
            )
        if j < _BPW - 1:
            pltpu.async_copy(val_v.at[j], out_hbm.at[b0 + j], osem)
    pltpu.sync_copy(val_v.at[_BPW - 1], out_hbm.at[b0 + _BPW - 1])
    for j in range(_BPW - 1):
        pltpu.make_async_copy(val_v.at[j], out_hbm.at[b0 + j], osem).wait()


def kernel(m, x, y):
    mruns = (
        m.reshape(8192, 8, 4, 128)
        .transpose(0, 2, 1, 3)
        .reshape(_B * _RUNS_PER_B, 128)
    )
    return _probe_gather(mruns, x.astype(jnp.int32), y.astype(jnp.int32))


# speedup vs baseline: 1.0697x; 1.0015x over previous
import functools

import jax
import jax.numpy as jnp
from jax import lax
from jax.experimental import pallas as pl
from jax.experimental.pallas import tpu as pltpu
from jax.experimental.pallas import tpu_sc as plsc

_B = 64
_P = 64
_H = 512
_W = 512

_L = 16
_NC = 1
_NS = 16
_NW = _NC * _NS
_BPW = _B // _NW

_RUNS_PER_B = 2 * _H * _W // 128

_mesh = plsc.VectorSubcoreMesh(core_axis_name="c", subcore_axis_name="s", num_cores=1)


@functools.partial(
    pl.kernel,
    mesh=_mesh,
    out_type=jax.ShapeDtypeStruct((_B, _P), jnp.float32),
    scratch_types=[
        pltpu.VMEM((2 * _P,), jnp.int32),
        pltpu.VMEM((_BPW, _P), jnp.int32),
        pltpu.VMEM((_BPW, _P, 128), jnp.float32),
        pltpu.VMEM((_BPW, _P), jnp.float32),
        pltpu.SemaphoreType.DMA,
        pltpu.SemaphoreType.DMA,
        pltpu.SemaphoreType.DMA,
        pltpu.SemaphoreType.DMA,
        pltpu.SemaphoreType.DMA,
    ],
    compiler_params=pltpu.CompilerParams(needs_layout_passes=False),
)
def _probe_gather(m_hbm, xy_hbm, out_hbm, xy_v, ridx_v, runs_v, val_v,
                  gsem0, gsem1, gsem2, gsem3, osem):
    wid = lax.axis_index("s")
    b0 = wid * _BPW
    pltpu.sync_copy(xy_hbm, xy_v)

    gsems = (gsem0, gsem1, gsem2, gsem3)
    gathers = []
    for j in range(_BPW):
        for c in range(_P // _L):
            xv = xy_v[pl.ds(c * _L, _L)]
            yv = xy_v[pl.ds(_P + c * _L, _L)]
            run = (xv >> 3) * 32 + (yv >> 7) * 8 + (xv & 7)
            ridx_v[j, pl.ds(c * _L, _L)] = run + (b0 + j) * _RUNS_PER_B
        gathers.append(
            pltpu.async_copy(m_hbm.at[ridx_v.at[j]], runs_v.at[j], gsems[j])
        )

    for j in range(_BPW):
        gathers[j].wait()
        for c in range(_P // _L):
            rids = lax.iota(jnp.int32, _L) + c * _L
            lids = xy_v[pl.ds(_P + c * _L, _L)] & 127
            val_v[j, pl.ds(c * _L, _L)] = plsc.load_gather(
                runs_v.at[j], [rids, lids]
            )
        if j < _BPW - 1:
            pltpu.async_copy(val_v.at[j], out_hbm.at[b0 + j], osem)
    pltpu.sync_copy(val_v.at[_BPW - 1], out_hbm.at[b0 + _BPW - 1])
    for j in range(_BPW - 1):
        pltpu.make_async_copy(val_v.at[j], out_hbm.at[b0 + j], osem).wait()


def kernel(m, x, y):
    mruns = (
        m.reshape(8192, 8, 4, 128)
        .transpose(0, 2, 1, 3)
        .reshape(_B * _RUNS_PER_B, 128)
    )
    xy = jnp.concatenate([x.astype(jnp.int32), y.astype(jnp.int32)])
    return _probe_gather(mruns, xy)
